# baseline (speedup 1.0000x reference)
import functools

import jax
import jax.numpy as jnp
from jax import lax
from jax.experimental import pallas as pl
from jax.experimental.pallas import tpu as pltpu
from jax.experimental.pallas import tpu_sc as plsc

_NC = 2
_NS = 16
_NW = _NC * _NS
_CHUNK = 128
_BC = 8192
_QB = 30
_Q = _QB * _BC


def _bf16_hi(x):
    u = lax.bitcast_convert_type(x, jnp.uint32)
    return (u + jnp.uint32(0x8000)) & jnp.uint32(0xFFFF0000)


def _repack_body(u1, u2, u3, u4, up):
    a = jnp.transpose(jnp.concatenate([u1[...], u3[...]], axis=0))
    b = jnp.transpose(jnp.concatenate([u2[...], u4[...]], axis=0))
    word = _bf16_hi(a) | (_bf16_hi(b) >> 16)
    up[...] = lax.bitcast_convert_type(word, jnp.float32)


def _repack(ut, nh):
    d, n = ut.shape
    grid = (pl.cdiv(nh, _BC),)
    specs = [pl.BlockSpec((d, _BC), lambda b, q=q: (0, b + q * _QB))
             for q in range(4)]
    out_spec = pl.BlockSpec((_BC, 2 * d), lambda b: (b, 0))
    return pl.pallas_call(
        _repack_body,
        grid=grid,
        in_specs=specs,
        out_specs=out_spec,
        out_shape=jax.ShapeDtypeStruct((nh, 2 * d), jnp.float32),
    )(ut, ut, ut, ut)


def _sc_gather_body(bpw, nch, w, idx_h, tab_h, out_h, idx, rows, sem):
    wid = lax.axis_index("s") * _NC + lax.axis_index("c")
    base = wid * bpw
    rbase = wid * nch
    pltpu.sync_copy(idx_h.at[pl.ds(rbase, nch)], idx)
    copies = [
        pltpu.async_copy(
            tab_h.at[idx.at[j]], rows.at[pl.ds(j * _CHUNK, _CHUNK)], sem)
        for j in range(nch)
    ]
    for c in copies:
        c.wait()
    pltpu.sync_copy(rows, out_h.at[pl.ds(base, bpw)])


def _sc_gather(idx2, tab):
    B = idx2.shape[0] * idx2.shape[1]
    w = tab.shape[1]
    bpw = B // _NW
    nch = bpw // _CHUNK
    mesh = plsc.VectorSubcoreMesh(core_axis_name="c", subcore_axis_name="s")
    kern = functools.partial(
        pl.kernel,
        out_type=jax.ShapeDtypeStruct((B, w), jnp.float32),
        mesh=mesh,
        scratch_types=[
            pltpu.VMEM((nch, _CHUNK), jnp.int32),
            pltpu.VMEM((bpw, w), jnp.float32),
            pltpu.SemaphoreType.DMA,
        ],
    )(functools.partial(_sc_gather_body, bpw, nch, w))
    return kern(idx2, tab)


def _unpack_select(g, q):
    d = g.shape[1] // 2
    u = lax.bitcast_convert_type(g, jnp.uint32)
    hi = lax.bitcast_convert_type(u & jnp.uint32(0xFFFF0000), jnp.float32)
    lo = lax.bitcast_convert_type(u << 16, jnp.float32)
    pick = jnp.where((q & 1) > 0, lo, hi)
    return jnp.where(q >= 2, pick[:, d:], pick[:, :d])


def _proj_body(gu_ref, uh_ref, w1a, b1, xu_ref):
    u = _unpack_select(gu_ref[...], uh_ref[...])
    xu_ref[...] = jnp.dot(
        u, w1a[...], preferred_element_type=jnp.float32) + b1[...]


def _proj(gu, uh, w1a, b1r):
    B, w = gu.shape
    d, H = w1a.shape
    T = 4096
    return pl.pallas_call(
        _proj_body,
        grid=(B // T,),
        in_specs=[
            pl.BlockSpec((T, w), lambda b: (b, 0)),
            pl.BlockSpec((T, 1), lambda b: (b, 0)),
            pl.BlockSpec((d, H), lambda b: (0, 0)),
            pl.BlockSpec((1, H), lambda b: (0, 0)),
        ],
        out_specs=pl.BlockSpec((T, H), lambda b: (b, 0)),
        out_shape=jax.ShapeDtypeStruct((B, H), jnp.float32),
    )(gu, uh, w1a, b1r)


def _mlp_body(xu_ref, gi_ref, ih_ref,
              w1b, w2, b2, w3, b3, o_ref):
    i = _unpack_select(gi_ref[...], ih_ref[...])
    h = xu_ref[...] + jnp.dot(
        i, w1b[...], preferred_element_type=jnp.float32)
    h = jnp.maximum(h, 0.0)
    h = jnp.maximum(
        jnp.dot(h, w2[...], preferred_element_type=jnp.float32) + b2[...], 0.0)
    o = jnp.dot(h, w3[...], preferred_element_type=jnp.float32) + b3[0, 0]
    o_ref[...] = o[:, 0]


def _mlp(xu, gi, ih, w1b, w2, b2r, w3, b3r):
    B, w = gi.shape
    d = w // 2
    H = w1b.shape[1]
    H2 = w2.shape[1]
    T = 4096
    return pl.pallas_call(
        _mlp_body,
        grid=(B // T,),
        in_specs=[
            pl.BlockSpec((T, H), lambda b: (b, 0)),
            pl.BlockSpec((T, w), lambda b: (b, 0)),
            pl.BlockSpec((T, 1), lambda b: (b, 0)),
            pl.BlockSpec((d, H), lambda b: (0, 0)),
            pl.BlockSpec((H, H2), lambda b: (0, 0)),
            pl.BlockSpec((1, H2), lambda b: (0, 0)),
            pl.BlockSpec((H2, 1), lambda b: (0, 0)),
            pl.BlockSpec((1, 1), lambda b: (0, 0)),
        ],
        out_specs=pl.BlockSpec((T,), lambda b: (b,)),
        out_shape=jax.ShapeDtypeStruct((B,), jnp.float32),
    )(xu, gi, ih, w1b, w2, b2r, w3, b3r)


def kernel(user, item, U, I, W1, b1, W2, b2, W3, b3):
    B = user.shape[0]
    N, D = U.shape
    nh = N - 3 * _Q
    user = user.astype(jnp.int32)
    item = item.astype(jnp.int32)
    uq = jnp.minimum(user // _Q, 3)
    iq = jnp.minimum(item // _Q, 3)
    up_idx = user - uq * _Q
    ip_idx = item - iq * _Q
    uh = uq.reshape(B, 1)
    ih = iq.reshape(B, 1)

    upk = _repack(U.T, nh)
    gu = _sc_gather(up_idx.reshape(B // _CHUNK, _CHUNK), upk)
    ipk = _repack(I.T, nh)
    gi = _sc_gather(ip_idx.reshape(B // _CHUNK, _CHUNK), ipk)
    w1a = W1[:D]
    w1b = W1[D:]
    xu = _proj(gu, uh, w1a, b1.reshape(1, -1))
    return _mlp(xu, gi, ih, w1b, W2, b2.reshape(1, -1), W3,
                b3.reshape(1, 1))


# speedup vs baseline: 1.0160x; 1.0160x over previous
import functools

import jax
import jax.numpy as jnp
from jax import lax
from jax.experimental import pallas as pl
from jax.experimental.pallas import tpu as pltpu
from jax.experimental.pallas import tpu_sc as plsc

_NC = 2
_NS = 16
_NW = _NC * _NS
_CHUNK = 128
_BC = 8192
_QB = 30
_Q = _QB * _BC


def _bf16_hi(x):
    u = lax.bitcast_convert_type(x, jnp.uint32)
    return (u + jnp.uint32(0x8000)) & jnp.uint32(0xFFFF0000)


def _repack_body(u1, u2, u3, u4, up):
    a = jnp.transpose(jnp.concatenate([u1[...], u3[...]], axis=0))
    b = jnp.transpose(jnp.concatenate([u2[...], u4[...]], axis=0))
    word = _bf16_hi(a) | (_bf16_hi(b) >> 16)
    up[...] = lax.bitcast_convert_type(word, jnp.float32)


def _repack(ut, nh):
    d, n = ut.shape
    grid = (pl.cdiv(nh, _BC),)
    specs = [pl.BlockSpec((d, _BC), lambda b, q=q: (0, b + q * _QB))
             for q in range(4)]
    out_spec = pl.BlockSpec((_BC, 2 * d), lambda b: (b, 0))
    return pl.pallas_call(
        _repack_body,
        grid=grid,
        in_specs=specs,
        out_specs=out_spec,
        out_shape=jax.ShapeDtypeStruct((nh, 2 * d), jnp.float32),
    )(ut, ut, ut, ut)


def _sc_gather_body(bpw, nch, w, idx_h, tab_h, out_h, idx, rows, sem):
    wid = lax.axis_index("s") * _NC + lax.axis_index("c")
    base = wid * bpw
    rbase = wid * nch
    pltpu.sync_copy(idx_h.at[pl.ds(rbase, nch)], idx)
    copies = [
        pltpu.async_copy(
            tab_h.at[idx.at[j]], rows.at[pl.ds(j * _CHUNK, _CHUNK)], sem)
        for j in range(nch)
    ]
    for c in copies:
        c.wait()
    pltpu.sync_copy(rows, out_h.at[pl.ds(base, bpw)])


def _sc_gather(idx2, tab):
    B = idx2.shape[0] * idx2.shape[1]
    w = tab.shape[1]
    bpw = B // _NW
    nch = bpw // _CHUNK
    mesh = plsc.VectorSubcoreMesh(core_axis_name="c", subcore_axis_name="s")
    kern = functools.partial(
        pl.kernel,
        out_type=jax.ShapeDtypeStruct((B, w), jnp.float32),
        mesh=mesh,
        scratch_types=[
            pltpu.VMEM((nch, _CHUNK), jnp.int32),
            pltpu.VMEM((bpw, w), jnp.float32),
            pltpu.SemaphoreType.DMA,
        ],
    )(functools.partial(_sc_gather_body, bpw, nch, w))
    return kern(idx2, tab)


def _unpack_select(g, q):
    d = g.shape[1] // 2
    u = lax.bitcast_convert_type(g, jnp.uint32)
    hi = lax.bitcast_convert_type(u & jnp.uint32(0xFFFF0000), jnp.float32)
    lo = lax.bitcast_convert_type(u << 16, jnp.float32)
    pick = jnp.where((q & 1) > 0, lo, hi)
    return jnp.where(q >= 2, pick[:, d:], pick[:, :d])


def _mlp_body(gu_ref, gi_ref, uh_ref, ih_ref,
              w1a, w1b, b1, w2, b2, w3, b3, o_ref):
    u = _unpack_select(gu_ref[...], uh_ref[...])
    i = _unpack_select(gi_ref[...], ih_ref[...])
    h = jnp.dot(u, w1a[...], preferred_element_type=jnp.float32)
    h = h + jnp.dot(i, w1b[...], preferred_element_type=jnp.float32)
    h = jnp.maximum(h + b1[...], 0.0)
    h = jnp.maximum(
        jnp.dot(h, w2[...], preferred_element_type=jnp.float32) + b2[...], 0.0)
    o = jnp.dot(h, w3[...], preferred_element_type=jnp.float32) + b3[0, 0]
    o_ref[...] = o[:, 0]


def _mlp(gu, gi, uh, ih, w1a, w1b, b1r, w2, b2r, w3, b3r):
    B, w = gu.shape
    d = w // 2
    H = w1a.shape[1]
    H2 = w2.shape[1]
    T = 4096
    return pl.pallas_call(
        _mlp_body,
        grid=(B // T,),
        in_specs=[
            pl.BlockSpec((T, w), lambda b: (b, 0)),
            pl.BlockSpec((T, w), lambda b: (b, 0)),
            pl.BlockSpec((T, 1), lambda b: (b, 0)),
            pl.BlockSpec((T, 1), lambda b: (b, 0)),
            pl.BlockSpec((d, H), lambda b: (0, 0)),
            pl.BlockSpec((d, H), lambda b: (0, 0)),
            pl.BlockSpec((1, H), lambda b: (0, 0)),
            pl.BlockSpec((H, H2), lambda b: (0, 0)),
            pl.BlockSpec((1, H2), lambda b: (0, 0)),
            pl.BlockSpec((H2, 1), lambda b: (0, 0)),
            pl.BlockSpec((1, 1), lambda b: (0, 0)),
        ],
        out_specs=pl.BlockSpec((T,), lambda b: (b,)),
        out_shape=jax.ShapeDtypeStruct((B,), jnp.float32),
    )(gu, gi, uh, ih, w1a, w1b, b1r, w2, b2r, w3, b3r)


def kernel(user, item, U, I, W1, b1, W2, b2, W3, b3):
    B = user.shape[0]
    N, D = U.shape
    nh = N - 3 * _Q
    user = user.astype(jnp.int32)
    item = item.astype(jnp.int32)
    uq = jnp.minimum(user // _Q, 3)
    iq = jnp.minimum(item // _Q, 3)
    up_idx = user - uq * _Q
    ip_idx = item - iq * _Q
    uh = uq.reshape(B, 1)
    ih = iq.reshape(B, 1)

    upk = _repack(U.T, nh)
    gu = _sc_gather(up_idx.reshape(B // _CHUNK, _CHUNK), upk)
    ipk = _repack(I.T, nh)
    gi = _sc_gather(ip_idx.reshape(B // _CHUNK, _CHUNK), ipk)
    w1a = W1[:D]
    w1b = W1[D:]
    return _mlp(gu, gi, uh, ih, w1a, w1b, b1.reshape(1, -1),
                W2, b2.reshape(1, -1), W3, b3.reshape(1, 1))


# speedup vs baseline: 1.0329x; 1.0167x over previous
import functools

import jax
import jax.numpy as jnp
from jax import lax
from jax.experimental import pallas as pl
from jax.experimental.pallas import tpu as pltpu
from jax.experimental.pallas import tpu_sc as plsc

_NC = 2
_NS = 16
_NW = _NC * _NS
_CHUNK = 128
_BC = 12288
_QB = 20
_Q = _QB * _BC


def _bf16_hi(x):
    u = lax.bitcast_convert_type(x, jnp.uint32)
    return (u + jnp.uint32(0x8000)) & jnp.uint32(0xFFFF0000)


def _repack_body(u1, u2, u3, u4, up):
    a = jnp.transpose(jnp.concatenate([u1[...], u3[...]], axis=0))
    b = jnp.transpose(jnp.concatenate([u2[...], u4[...]], axis=0))
    word = _bf16_hi(a) | (_bf16_hi(b) >> 16)
    up[...] = lax.bitcast_convert_type(word, jnp.float32)


def _repack(ut, nh):
    d, n = ut.shape
    grid = (pl.cdiv(nh, _BC),)
    specs = [pl.BlockSpec((d, _BC), lambda b, q=q: (0, b + q * _QB))
             for q in range(4)]
    out_spec = pl.BlockSpec((_BC, 2 * d), lambda b: (b, 0))
    return pl.pallas_call(
        _repack_body,
        grid=grid,
        in_specs=specs,
        out_specs=out_spec,
        out_shape=jax.ShapeDtypeStruct((nh, 2 * d), jnp.float32),
    )(ut, ut, ut, ut)


def _sc_gather_body(bpw, nch, w, idx_h, tab_h, out_h, idx, rows, sem):
    wid = lax.axis_index("s") * _NC + lax.axis_index("c")
    base = wid * bpw
    rbase = wid * nch
    pltpu.sync_copy(idx_h.at[pl.ds(rbase, nch)], idx)
    copies = [
        pltpu.async_copy(
            tab_h.at[idx.at[j]], rows.at[pl.ds(j * _CHUNK, _CHUNK)], sem)
        for j in range(nch)
    ]
    for c in copies:
        c.wait()
    pltpu.sync_copy(rows, out_h.at[pl.ds(base, bpw)])


def _sc_gather(idx2, tab):
    B = idx2.shape[0] * idx2.shape[1]
    w = tab.shape[1]
    bpw = B // _NW
    nch = bpw // _CHUNK
    mesh = plsc.VectorSubcoreMesh(core_axis_name="c", subcore_axis_name="s")
    kern = functools.partial(
        pl.kernel,
        out_type=jax.ShapeDtypeStruct((B, w), jnp.float32),
        mesh=mesh,
        scratch_types=[
            pltpu.VMEM((nch, _CHUNK), jnp.int32),
            pltpu.VMEM((bpw, w), jnp.float32),
            pltpu.SemaphoreType.DMA,
        ],
    )(functools.partial(_sc_gather_body, bpw, nch, w))
    return kern(idx2, tab)


def _unpack_select(g, q):
    d = g.shape[1] // 2
    u = lax.bitcast_convert_type(g, jnp.uint32)
    hi = lax.bitcast_convert_type(u & jnp.uint32(0xFFFF0000), jnp.float32)
    lo = lax.bitcast_convert_type(u << 16, jnp.float32)
    pick = jnp.where((q & 1) > 0, lo, hi)
    return jnp.where(q >= 2, pick[:, d:], pick[:, :d])


def _mlp_body(gu_ref, gi_ref, uh_ref, ih_ref,
              w1a, w1b, b1, w2, b2, w3, b3, o_ref):
    u = _unpack_select(gu_ref[...], uh_ref[...])
    i = _unpack_select(gi_ref[...], ih_ref[...])
    h = jnp.dot(u, w1a[...], preferred_element_type=jnp.float32)
    h = h + jnp.dot(i, w1b[...], preferred_element_type=jnp.float32)
    h = jnp.maximum(h + b1[...], 0.0)
    h = jnp.maximum(
        jnp.dot(h, w2[...], preferred_element_type=jnp.float32) + b2[...], 0.0)
    o = jnp.dot(h, w3[...], preferred_element_type=jnp.float32) + b3[0, 0]
    o_ref[...] = o[:, 0]


def _mlp(gu, gi, uh, ih, w1a, w1b, b1r, w2, b2r, w3, b3r):
    B, w = gu.shape
    d = w // 2
    H = w1a.shape[1]
    H2 = w2.shape[1]
    T = 4096
    return pl.pallas_call(
        _mlp_body,
        grid=(B // T,),
        in_specs=[
            pl.BlockSpec((T, w), lambda b: (b, 0)),
            pl.BlockSpec((T, w), lambda b: (b, 0)),
            pl.BlockSpec((T, 1), lambda b: (b, 0)),
            pl.BlockSpec((T, 1), lambda b: (b, 0)),
            pl.BlockSpec((d, H), lambda b: (0, 0)),
            pl.BlockSpec((d, H), lambda b: (0, 0)),
            pl.BlockSpec((1, H), lambda b: (0, 0)),
            pl.BlockSpec((H, H2), lambda b: (0, 0)),
            pl.BlockSpec((1, H2), lambda b: (0, 0)),
            pl.BlockSpec((H2, 1), lambda b: (0, 0)),
            pl.BlockSpec((1, 1), lambda b: (0, 0)),
        ],
        out_specs=pl.BlockSpec((T,), lambda b: (b,)),
        out_shape=jax.ShapeDtypeStruct((B,), jnp.float32),
    )(gu, gi, uh, ih, w1a, w1b, b1r, w2, b2r, w3, b3r)


def kernel(user, item, U, I, W1, b1, W2, b2, W3, b3):
    B = user.shape[0]
    N, D = U.shape
    nh = N - 3 * _Q
    user = user.astype(jnp.int32)
    item = item.astype(jnp.int32)
    uq = jnp.minimum(user // _Q, 3)
    iq = jnp.minimum(item // _Q, 3)
    up_idx = user - uq * _Q
    ip_idx = item - iq * _Q
    uh = uq.reshape(B, 1)
    ih = iq.reshape(B, 1)

    upk = _repack(U.T, nh)
    gu = _sc_gather(up_idx.reshape(B // _CHUNK, _CHUNK), upk)
    ipk = _repack(I.T, nh)
    gi = _sc_gather(ip_idx.reshape(B // _CHUNK, _CHUNK), ipk)
    w1a = W1[:D]
    w1b = W1[D:]
    return _mlp(gu, gi, uh, ih, w1a, w1b, b1.reshape(1, -1),
                W2, b2.reshape(1, -1), W3, b3.reshape(1, 1))
